# baseline (speedup 1.0000x reference)
import jax
import jax.numpy as jnp
from jax.experimental import pallas as pl
from jax.experimental.pallas import tpu as pltpu

_N = 16384
_D = 64
_K = 32
_BM = 128
_CW = 2048
_GW = 256
_NG = _N // _GW
_GSH = 8
_T = 7
_INF = jnp.inf


def _knn_block_kernel(xb_ref, xt_ref, c_ref, i_ref, dscr_ref):
    i = pl.program_id(0)
    xb = xb_ref[...]
    sq_row = jnp.sum(xb * xb, axis=1, keepdims=True)
    gv_l = [[] for _ in range(_T)]
    gi_l = [[] for _ in range(_T)]
    ng_c = _CW // _GW
    for cc in range(_N // _CW):
        xt_c = xt_ref[:, cc * _CW:(cc + 1) * _CW]
        sq_c = jnp.sum(xt_c * xt_c, axis=0, keepdims=True)
        d = sq_row + sq_c - 2.0 * jnp.dot(
            xb, xt_c, preferred_element_type=jnp.float32)
        colc = jax.lax.broadcasted_iota(jnp.int32, (_BM, _CW), 1) + cc * _CW
        rowc = jax.lax.broadcasted_iota(jnp.int32, (_BM, _CW), 0) + i * _BM
        d = jnp.where(colc == rowc, _INF, d)
        dscr_ref[:, cc * _CW:(cc + 1) * _CW] = d
        m = d.reshape(_BM, ng_c, _GW)
        c3 = colc.reshape(_BM, ng_c, _GW)
        for lvl in range(_T):
            av = jnp.min(m, axis=2)
            ai = jnp.min(jnp.where(m == av[:, :, None], c3, _N), axis=2)
            gv_l[lvl].append(av)
            gi_l[lvl].append(ai)
            if lvl + 1 < _T:
                m = jnp.where(c3 == ai[:, :, None], _INF, m)
    hv = [jnp.concatenate(gv_l[t], axis=1) for t in range(_T)]
    hi = [jnp.concatenate(gi_l[t], axis=1) for t in range(_T)]

    kcols = jax.lax.broadcasted_iota(jnp.int32, (_BM, _K), 1)
    giota = jax.lax.broadcasted_iota(jnp.int32, (_BM, _NG), 1)

    def body(j, carry):
        hv, hi, cacc, iacc = carry
        w = hv[0]
        mv = jnp.min(w, axis=1, keepdims=True)
        gsel = jnp.min(jnp.where(w == mv, giota, _NG), axis=1, keepdims=True)
        onehot = giota == gsel
        mi = jnp.sum(jnp.where(onehot, hi[0], 0), axis=1, keepdims=True)
        cacc = jnp.where(kcols == j, mv, cacc)
        iacc = jnp.where(kcols == j, mi, iacc)
        hv = [jnp.where(onehot, hv[t + 1], hv[t]) for t in range(_T - 1)] + [
            jnp.where(onehot, _INF, hv[_T - 1])]
        hi = [jnp.where(onehot, hi[t + 1], hi[t]) for t in range(_T - 1)] + [
            jnp.where(onehot, _N, hi[_T - 1])]
        return hv, hi, cacc, iacc

    cacc0 = jnp.zeros((_BM, _K), jnp.float32)
    iacc0 = jnp.zeros((_BM, _K), jnp.int32)
    _, _, cacc, iacc = jax.lax.fori_loop(0, _K, body, (hv, hi, cacc0, iacc0), unroll=_K)

    giota3 = jax.lax.broadcasted_iota(jnp.int32, (_BM, _K, _NG), 2)
    grp_sel = jax.lax.shift_right_logical(iacc, _GSH)
    cnt_sel = jnp.sum((grp_sel[:, :, None] == giota3).astype(jnp.int32),
                      axis=1)
    frow = jnp.any(cnt_sel >= _T, axis=1, keepdims=True)

    def fallback(carry):
        cacc2, iacc2 = carry

        def fb_body(j, c2):
            ca, ia, pv, pi = c2
            mv = jnp.full((_BM, 1), _INF, jnp.float32)
            mi = jnp.full((_BM, 1), _N, jnp.int32)
            for cc in range(_N // _CW):
                dc = dscr_ref[:, cc * _CW:(cc + 1) * _CW]
                colc = (jax.lax.broadcasted_iota(jnp.int32, (_BM, _CW), 1)
                        + cc * _CW)
                validc = (dc > pv) | ((dc == pv) & (colc > pi))
                tv = jnp.min(jnp.where(validc, dc, _INF),
                             axis=1, keepdims=True)
                ti = jnp.min(jnp.where(validc & (dc == tv), colc, _N),
                             axis=1, keepdims=True)
                better = (tv < mv) | ((tv == mv) & (ti < mi))
                mv = jnp.where(better, tv, mv)
                mi = jnp.where(better, ti, mi)
            ca = jnp.where((kcols == j) & frow, mv, ca)
            ia = jnp.where((kcols == j) & frow, mi, ia)
            return ca, ia, mv, mi

        pv0 = jnp.full((_BM, 1), -_INF, jnp.float32)
        pi0 = jnp.full((_BM, 1), -1, jnp.int32)
        ca, ia, _, _ = jax.lax.fori_loop(
            0, _K, fb_body, (cacc2, iacc2, pv0, pi0))
        return ca, ia

    cacc, iacc = jax.lax.cond(
        jnp.any(frow), fallback, lambda c: c, (cacc, iacc))
    c_ref[...] = cacc
    i_ref[...] = iacc


def kernel(X, k):
    del k
    xt = X.T
    grid = (_N // _BM,)
    c, idx = pl.pallas_call(
        _knn_block_kernel,
        grid=grid,
        in_specs=[
            pl.BlockSpec((_BM, _D), lambda i: (i, 0)),
            pl.BlockSpec((_D, _N), lambda i: (0, 0)),
        ],
        out_specs=[
            pl.BlockSpec((_BM, _K), lambda i: (i, 0)),
            pl.BlockSpec((_BM, _K), lambda i: (i, 0)),
        ],
        out_shape=[
            jax.ShapeDtypeStruct((_N, _K), jnp.float32),
            jax.ShapeDtypeStruct((_N, _K), jnp.int32),
        ],
        scratch_shapes=[pltpu.VMEM((_BM, _N), jnp.float32)],
    )(X, xt)
    return c, idx


# speedup vs baseline: 1.2370x; 1.2370x over previous
import jax
import jax.numpy as jnp
from jax.experimental import pallas as pl
from jax.experimental.pallas import tpu as pltpu

_N = 16384
_D = 64
_K = 32
_BM = 128
_CW = 2048
_GW = 256
_NG = _N // _GW
_GSH = 8
_T = 7
_INF = jnp.inf


def _knn_block_kernel(xb_ref, xt_ref, c_ref, i_ref, dscr_ref):
    i = pl.program_id(0)
    xb = xb_ref[...]
    sq_row = jnp.sum(xb * xb, axis=1, keepdims=True)
    gv_l = [[] for _ in range(_T)]
    gi_l = [[] for _ in range(_T)]
    ng_c = _CW // _GW
    for cc in range(_N // _CW):
        xt_c = xt_ref[:, cc * _CW:(cc + 1) * _CW]
        sq_c = jnp.sum(xt_c * xt_c, axis=0, keepdims=True)
        d = sq_row + sq_c - 2.0 * jnp.dot(
            xb, xt_c, preferred_element_type=jnp.float32)
        colc = jax.lax.broadcasted_iota(jnp.int32, (_BM, _CW), 1) + cc * _CW
        rowc = jax.lax.broadcasted_iota(jnp.int32, (_BM, _CW), 0) + i * _BM
        d = jnp.where(colc == rowc, _INF, d)
        dscr_ref[:, cc * _CW:(cc + 1) * _CW] = d
        m = d.reshape(_BM, ng_c, _GW)
        c3 = colc.reshape(_BM, ng_c, _GW)
        for lvl in range(_T):
            av = jnp.min(m, axis=2)
            ai = jnp.min(jnp.where(m == av[:, :, None], c3, _N), axis=2)
            gv_l[lvl].append(av)
            gi_l[lvl].append(ai)
            if lvl + 1 < _T:
                m = jnp.where(c3 == ai[:, :, None], _INF, m)
    hv = [jnp.concatenate(gv_l[t], axis=1) for t in range(_T)]
    hi = [jnp.concatenate(gi_l[t], axis=1) for t in range(_T)]

    kcols = jax.lax.broadcasted_iota(jnp.int32, (_BM, _K), 1)
    giota = jax.lax.broadcasted_iota(jnp.int32, (_BM, _NG), 1)

    def body(j, carry):
        hv, hi, cacc, iacc = carry
        w = hv[0]
        mv = jnp.min(w, axis=1, keepdims=True)
        gsel = jnp.min(jnp.where(w == mv, giota, _NG), axis=1, keepdims=True)
        onehot = giota == gsel
        mi = jnp.sum(jnp.where(onehot, hi[0], 0), axis=1, keepdims=True)
        cacc = jnp.where(kcols == j, mv, cacc)
        iacc = jnp.where(kcols == j, mi, iacc)
        hv = [jnp.where(onehot, hv[t + 1], hv[t]) for t in range(_T - 1)] + [
            jnp.where(onehot, _INF, hv[_T - 1])]
        hi = [jnp.where(onehot, hi[t + 1], hi[t]) for t in range(_T - 1)] + [
            jnp.where(onehot, _N, hi[_T - 1])]
        return hv, hi, cacc, iacc

    cacc0 = jnp.zeros((_BM, _K), jnp.float32)
    iacc0 = jnp.zeros((_BM, _K), jnp.int32)
    _, _, cacc, iacc = jax.lax.fori_loop(0, _K, body, (hv, hi, cacc0, iacc0), unroll=8)

    giota3 = jax.lax.broadcasted_iota(jnp.int32, (_BM, _K, _NG), 2)
    grp_sel = jax.lax.shift_right_logical(iacc, _GSH)
    cnt_sel = jnp.sum((grp_sel[:, :, None] == giota3).astype(jnp.int32),
                      axis=1)
    frow = jnp.any(cnt_sel >= _T, axis=1, keepdims=True)

    def fallback(carry):
        cacc2, iacc2 = carry

        def fb_body(j, c2):
            ca, ia, pv, pi = c2
            mv = jnp.full((_BM, 1), _INF, jnp.float32)
            mi = jnp.full((_BM, 1), _N, jnp.int32)
            for cc in range(_N // _CW):
                dc = dscr_ref[:, cc * _CW:(cc + 1) * _CW]
                colc = (jax.lax.broadcasted_iota(jnp.int32, (_BM, _CW), 1)
                        + cc * _CW)
                validc = (dc > pv) | ((dc == pv) & (colc > pi))
                tv = jnp.min(jnp.where(validc, dc, _INF),
                             axis=1, keepdims=True)
                ti = jnp.min(jnp.where(validc & (dc == tv), colc, _N),
                             axis=1, keepdims=True)
                better = (tv < mv) | ((tv == mv) & (ti < mi))
                mv = jnp.where(better, tv, mv)
                mi = jnp.where(better, ti, mi)
            ca = jnp.where((kcols == j) & frow, mv, ca)
            ia = jnp.where((kcols == j) & frow, mi, ia)
            return ca, ia, mv, mi

        pv0 = jnp.full((_BM, 1), -_INF, jnp.float32)
        pi0 = jnp.full((_BM, 1), -1, jnp.int32)
        ca, ia, _, _ = jax.lax.fori_loop(
            0, _K, fb_body, (cacc2, iacc2, pv0, pi0))
        return ca, ia

    cacc, iacc = jax.lax.cond(
        jnp.any(frow), fallback, lambda c: c, (cacc, iacc))
    c_ref[...] = cacc
    i_ref[...] = iacc


def kernel(X, k):
    del k
    xt = X.T
    grid = (_N // _BM,)
    c, idx = pl.pallas_call(
        _knn_block_kernel,
        grid=grid,
        in_specs=[
            pl.BlockSpec((_BM, _D), lambda i: (i, 0)),
            pl.BlockSpec((_D, _N), lambda i: (0, 0)),
        ],
        out_specs=[
            pl.BlockSpec((_BM, _K), lambda i: (i, 0)),
            pl.BlockSpec((_BM, _K), lambda i: (i, 0)),
        ],
        out_shape=[
            jax.ShapeDtypeStruct((_N, _K), jnp.float32),
            jax.ShapeDtypeStruct((_N, _K), jnp.int32),
        ],
        scratch_shapes=[pltpu.VMEM((_BM, _N), jnp.float32)],
    )(X, xt)
    return c, idx
